# baseline (speedup 1.0000x reference)
import numpy as np

import jax
import jax.numpy as jnp
from jax.experimental import pallas as pl
from jax.experimental.pallas import tpu as pltpu

_NUM_TRAIN_TIMESTEPS = 1000
_BETA_START = 0.0001
_BETA_END = 0.02


def _schedule_tables():
    betas = np.linspace(_BETA_START, _BETA_END, _NUM_TRAIN_TIMESTEPS,
                        dtype=np.float32)
    alphas_cumprod = np.cumprod(1.0 - betas, axis=0, dtype=np.float32)
    sqrt_a = np.sqrt(alphas_cumprod).astype(np.float32)
    sqrt_oma = np.sqrt(1.0 - alphas_cumprod).astype(np.float32)
    return sqrt_a, sqrt_oma


_SQRT_A, _SQRT_OMA = _schedule_tables()


def _fma_body(ts_ref, ta_ref, tb_ref, x_ref, n_ref, o_ref):
    b = pl.program_id(0)
    t = ts_ref[b]
    a = ta_ref[t]
    s = tb_ref[t]
    o_ref[...] = x_ref[...] * a + n_ref[...] * s


_BN = 256


def kernel(x_0, noise, timesteps):
    B, N, C = x_0.shape
    grid = (B, N // _BN)
    spec = pl.BlockSpec((1, _BN, C), lambda b, j, *_: (b, j, 0))
    return pl.pallas_call(
        _fma_body,
        grid_spec=pltpu.PrefetchScalarGridSpec(
            num_scalar_prefetch=3,
            grid=grid,
            in_specs=[spec, spec],
            out_specs=spec,
        ),
        out_shape=jax.ShapeDtypeStruct((B, N, C), x_0.dtype),
    )(timesteps.astype(jnp.int32), jnp.asarray(_SQRT_A), jnp.asarray(_SQRT_OMA),
      x_0, noise)


# speedup vs baseline: 1.1451x; 1.1451x over previous
import numpy as np

import jax
import jax.numpy as jnp
from jax.experimental import pallas as pl
from jax.experimental.pallas import tpu as pltpu

_NUM_TRAIN_TIMESTEPS = 1000
_BETA_START = 0.0001
_BETA_END = 0.02


def _schedule_tables():
    betas = np.linspace(_BETA_START, _BETA_END, _NUM_TRAIN_TIMESTEPS,
                        dtype=np.float32)
    alphas_cumprod = np.cumprod(1.0 - betas, axis=0, dtype=np.float32)
    sqrt_a = np.sqrt(alphas_cumprod).astype(np.float32)
    sqrt_oma = np.sqrt(1.0 - alphas_cumprod).astype(np.float32)
    return sqrt_a, sqrt_oma


_SQRT_A, _SQRT_OMA = _schedule_tables()


def _fma_body(ts_ref, ta_ref, tb_ref, x_ref, n_ref, o_ref):
    b = pl.program_id(0)
    t = ts_ref[b]
    a = ta_ref[t]
    s = tb_ref[t]
    o_ref[...] = x_ref[...] * a + n_ref[...] * s


_BN = 512


def kernel(x_0, noise, timesteps):
    B, N, C = x_0.shape
    grid = (B, N // _BN)
    spec = pl.BlockSpec((1, _BN, C), lambda b, j, *_: (b, j, 0))
    return pl.pallas_call(
        _fma_body,
        grid_spec=pltpu.PrefetchScalarGridSpec(
            num_scalar_prefetch=3,
            grid=grid,
            in_specs=[spec, spec],
            out_specs=spec,
        ),
        out_shape=jax.ShapeDtypeStruct((B, N, C), x_0.dtype),
        compiler_params=pltpu.CompilerParams(
            dimension_semantics=("parallel", "parallel"),
        ),
    )(timesteps.astype(jnp.int32), jnp.asarray(_SQRT_A), jnp.asarray(_SQRT_OMA),
      x_0, noise)


# speedup vs baseline: 1.1648x; 1.0172x over previous
import numpy as np

import jax
import jax.numpy as jnp
from jax.experimental import pallas as pl
from jax.experimental.pallas import tpu as pltpu

_NUM_TRAIN_TIMESTEPS = 1000
_BETA_START = 0.0001
_BETA_END = 0.02


def _schedule_tables():
    betas = np.linspace(_BETA_START, _BETA_END, _NUM_TRAIN_TIMESTEPS,
                        dtype=np.float32)
    alphas_cumprod = np.cumprod(1.0 - betas, axis=0, dtype=np.float32)
    sqrt_a = np.sqrt(alphas_cumprod).astype(np.float32)
    sqrt_oma = np.sqrt(1.0 - alphas_cumprod).astype(np.float32)
    return sqrt_a, sqrt_oma


_SQRT_A, _SQRT_OMA = _schedule_tables()


def _fma_body(ts_ref, ta_ref, tb_ref, x_ref, n_ref, o_ref):
    b = pl.program_id(0)
    t = ts_ref[b]
    a = ta_ref[t]
    s = tb_ref[t]
    o_ref[...] = x_ref[...] * a + n_ref[...] * s


_BN = 1024


def kernel(x_0, noise, timesteps):
    B, N, C = x_0.shape
    grid = (B, N // _BN)
    spec = pl.BlockSpec((1, _BN, C), lambda b, j, *_: (b, j, 0))
    return pl.pallas_call(
        _fma_body,
        grid_spec=pltpu.PrefetchScalarGridSpec(
            num_scalar_prefetch=3,
            grid=grid,
            in_specs=[spec, spec],
            out_specs=spec,
        ),
        out_shape=jax.ShapeDtypeStruct((B, N, C), x_0.dtype),
        compiler_params=pltpu.CompilerParams(
            dimension_semantics=("parallel", "parallel"),
        ),
    )(timesteps.astype(jnp.int32), jnp.asarray(_SQRT_A), jnp.asarray(_SQRT_OMA),
      x_0, noise)
